# baseline (device time: 50592 ns/iter reference)
import jax
import jax.numpy as jnp
from jax import lax
from jax.experimental import pallas as pl
from jax.experimental.pallas import tpu as pltpu

NCHUNK = 4


def kernel(O, Wo):
    B, S, H, D = O.shape
    K = H * D
    N = Wo.shape[1]
    S_out = S // 2
    Q = S_out // 2
    SPLIT = NCHUNK // B
    R = Q // SPLIT

    OT = O.transpose(0, 2, 3, 1).reshape(B, K, S)

    def body(o_ref, wo_ref, out_ref, wo_bf, o_vm, x_send, x_recv, y_send,
             y_recv, acc, in_dma_sems, x_send_sems, x_recv_sems,
             y_send_sems, y_recv_sems):
        my_x = lax.axis_index("x")
        my_y = lax.axis_index("y")
        peer_x = 1 - my_x
        peer_y = 1 - my_y

        def chunk(c):
            return c // SPLIT, (c % SPLIT) * R

        in_dmas = []
        for b in range(B):
            dma = pltpu.make_async_copy(
                o_ref.at[b], o_vm.at[b], in_dma_sems.at[b],
            )
            dma.start()
            in_dmas.append(dma)

        barrier = pltpu.get_barrier_semaphore()
        pl.semaphore_signal(
            barrier, inc=1, device_id=(peer_x, my_y),
            device_id_type=pl.DeviceIdType.MESH,
        )
        pl.semaphore_signal(
            barrier, inc=1, device_id=(my_x, peer_y),
            device_id_type=pl.DeviceIdType.MESH,
        )
        pl.semaphore_wait(barrier, 2)

        wo_bf[...] = wo_ref[...].astype(jnp.bfloat16)

        def dot_t(a_t):
            return lax.dot_general(
                a_t.astype(jnp.bfloat16), wo_bf[...],
                dimension_numbers=(((0,), (0,)), ((), ())),
                preferred_element_type=jnp.float32,
            )

        waited = [False] * B
        x_rdmas = []
        for c in range(NCHUNK):
            b, r = chunk(c)
            if not waited[b]:
                in_dmas[b].wait()
                waited[b] = True
            x_send[c, :, :] = dot_t(
                o_vm[b, :, pl.ds(peer_x * S_out + my_y * Q + r, R)]
            ).astype(jnp.bfloat16)
            rdma = pltpu.make_async_remote_copy(
                src_ref=x_send.at[c],
                dst_ref=x_recv.at[c],
                send_sem=x_send_sems.at[c],
                recv_sem=x_recv_sems.at[c],
                device_id=(peer_x, my_y),
                device_id_type=pl.DeviceIdType.MESH,
            )
            rdma.start()
            x_rdmas.append(rdma)

        for c in range(NCHUNK):
            b, r = chunk(c)
            acc[b, pl.ds(r, R), :] = dot_t(
                o_vm[b, :, pl.ds(my_x * S_out + my_y * Q + r, R)]
            )

        y_rdmas = []
        for c in range(NCHUNK):
            b, r = chunk(c)
            x_rdmas[c].wait_recv()
            s = acc[b, pl.ds(r, R), :] + x_recv[c, :, :].astype(jnp.float32)
            out_ref[b, pl.ds(my_y * Q + r, R), :] = s
            y_send[c, :, :] = s.astype(jnp.bfloat16)
            rdma = pltpu.make_async_remote_copy(
                src_ref=y_send.at[c],
                dst_ref=y_recv.at[c],
                send_sem=y_send_sems.at[c],
                recv_sem=y_recv_sems.at[c],
                device_id=(my_x, peer_y),
                device_id_type=pl.DeviceIdType.MESH,
            )
            rdma.start()
            y_rdmas.append(rdma)

        for c in range(NCHUNK):
            b, r = chunk(c)
            y_rdmas[c].wait_recv()
            out_ref[b, pl.ds(peer_y * Q + r, R), :] = (
                y_recv[c, :, :].astype(jnp.float32)
            )

        for c in range(NCHUNK):
            x_rdmas[c].wait_send()
            y_rdmas[c].wait_send()

    return pl.pallas_call(
        body,
        out_shape=jax.ShapeDtypeStruct((B, S_out, N), jnp.float32),
        in_specs=[
            pl.BlockSpec(memory_space=pltpu.MemorySpace.HBM),
            pl.BlockSpec(memory_space=pltpu.VMEM),
        ],
        out_specs=pl.BlockSpec(memory_space=pltpu.VMEM),
        scratch_shapes=[
            pltpu.VMEM((K, N), jnp.bfloat16),
            pltpu.VMEM((B, K, S), jnp.float32),
            pltpu.VMEM((NCHUNK, R, N), jnp.bfloat16),
            pltpu.VMEM((NCHUNK, R, N), jnp.bfloat16),
            pltpu.VMEM((NCHUNK, R, N), jnp.bfloat16),
            pltpu.VMEM((NCHUNK, R, N), jnp.bfloat16),
            pltpu.VMEM((B, Q, N), jnp.float32),
            pltpu.SemaphoreType.DMA((B,)),
            pltpu.SemaphoreType.DMA((NCHUNK,)),
            pltpu.SemaphoreType.DMA((NCHUNK,)),
            pltpu.SemaphoreType.DMA((NCHUNK,)),
            pltpu.SemaphoreType.DMA((NCHUNK,)),
        ],
        compiler_params=pltpu.CompilerParams(
            collective_id=0, vmem_limit_bytes=64 * 1024 * 1024,
        ),
    )(OT, Wo)


# device time: 45601 ns/iter; 1.1094x vs baseline; 1.1094x over previous
import jax
import jax.numpy as jnp
from jax import lax
from jax.experimental import pallas as pl
from jax.experimental.pallas import tpu as pltpu

NCHUNK = 4


def kernel(O, Wo):
    B, S, H, D = O.shape
    K = H * D
    N = Wo.shape[1]
    S_out = S // 2
    Q = S_out // 2
    SPLIT = NCHUNK // B
    R = Q // SPLIT

    OT = O.transpose(0, 2, 3, 1).reshape(B, K, S)

    def body(o_ref, wo_ref, out_ref, wo_bf, x_send, x_recv, y_send, y_recv,
             x_send_sems, x_recv_sems, y_send_sems, y_recv_sems):
        my_x = lax.axis_index("x")
        my_y = lax.axis_index("y")
        peer_x = 1 - my_x
        peer_y = 1 - my_y

        barrier = pltpu.get_barrier_semaphore()
        pl.semaphore_signal(
            barrier, inc=1, device_id=(peer_x, my_y),
            device_id_type=pl.DeviceIdType.MESH,
        )
        pl.semaphore_signal(
            barrier, inc=1, device_id=(my_x, peer_y),
            device_id_type=pl.DeviceIdType.MESH,
        )
        pl.semaphore_wait(barrier, 2)

        wo_bf[...] = wo_ref[...].astype(jnp.bfloat16)

        def chunk(c):
            return c // SPLIT, (c % SPLIT) * R

        def dot_t(a_t):
            return lax.dot_general(
                a_t.astype(jnp.bfloat16), wo_bf[...],
                dimension_numbers=(((0,), (0,)), ((), ())),
                preferred_element_type=jnp.float32,
            )

        x_rdmas = []
        for c in range(NCHUNK):
            b, r = chunk(c)
            x_send[c, :, :] = dot_t(
                o_ref[b, :, pl.ds(peer_x * S_out + my_y * Q + r, R)]
            ).astype(jnp.bfloat16)
            rdma = pltpu.make_async_remote_copy(
                src_ref=x_send.at[c],
                dst_ref=x_recv.at[c],
                send_sem=x_send_sems.at[c],
                recv_sem=x_recv_sems.at[c],
                device_id=(peer_x, my_y),
                device_id_type=pl.DeviceIdType.MESH,
            )
            rdma.start()
            x_rdmas.append(rdma)

        for c in range(NCHUNK):
            b, r = chunk(c)
            out_ref[b, pl.ds(my_y * Q + r, R), :] = dot_t(
                o_ref[b, :, pl.ds(my_x * S_out + my_y * Q + r, R)]
            )

        y_rdmas = []
        for c in range(NCHUNK):
            b, r = chunk(c)
            x_rdmas[c].wait_recv()
            sl = pl.ds(my_y * Q + r, R)
            s = out_ref[b, sl, :] + x_recv[c, :, :].astype(jnp.float32)
            out_ref[b, sl, :] = s
            y_send[c, :, :] = s.astype(jnp.bfloat16)
            rdma = pltpu.make_async_remote_copy(
                src_ref=y_send.at[c],
                dst_ref=y_recv.at[c],
                send_sem=y_send_sems.at[c],
                recv_sem=y_recv_sems.at[c],
                device_id=(my_x, peer_y),
                device_id_type=pl.DeviceIdType.MESH,
            )
            rdma.start()
            y_rdmas.append(rdma)

        for c in range(NCHUNK):
            b, r = chunk(c)
            y_rdmas[c].wait_recv()
            out_ref[b, pl.ds(peer_y * Q + r, R), :] = (
                y_recv[c, :, :].astype(jnp.float32)
            )

        for c in range(NCHUNK):
            x_rdmas[c].wait_send()
            y_rdmas[c].wait_send()

    return pl.pallas_call(
        body,
        out_shape=jax.ShapeDtypeStruct((B, S_out, N), jnp.float32),
        in_specs=[
            pl.BlockSpec(memory_space=pltpu.VMEM),
            pl.BlockSpec(memory_space=pltpu.VMEM),
        ],
        out_specs=pl.BlockSpec(memory_space=pltpu.VMEM),
        scratch_shapes=[
            pltpu.VMEM((K, N), jnp.bfloat16),
            pltpu.VMEM((NCHUNK, R, N), jnp.bfloat16),
            pltpu.VMEM((NCHUNK, R, N), jnp.bfloat16),
            pltpu.VMEM((NCHUNK, R, N), jnp.bfloat16),
            pltpu.VMEM((NCHUNK, R, N), jnp.bfloat16),
            pltpu.SemaphoreType.DMA((NCHUNK,)),
            pltpu.SemaphoreType.DMA((NCHUNK,)),
            pltpu.SemaphoreType.DMA((NCHUNK,)),
            pltpu.SemaphoreType.DMA((NCHUNK,)),
        ],
        compiler_params=pltpu.CompilerParams(collective_id=0),
    )(OT, Wo)


# device time: 45583 ns/iter; 1.1099x vs baseline; 1.0004x over previous
import jax
import jax.numpy as jnp
from jax import lax
from jax.experimental import pallas as pl
from jax.experimental.pallas import tpu as pltpu

NCHUNK = 4


def kernel(O, Wo):
    B, S, H, D = O.shape
    K = H * D
    N = Wo.shape[1]
    S_out = S // 2
    Q = S_out // 2
    SPLIT = NCHUNK // B
    R = Q // SPLIT

    OT = O.transpose(0, 2, 3, 1).reshape(B, K, S)

    def body(o_ref, wo_ref, out_ref, wo_bf, x_send, x_recv, y_send, y_recv,
             x_send_sems, x_recv_sems, y_send_sems, y_recv_sems):
        my_x = lax.axis_index("x")
        my_y = lax.axis_index("y")
        peer_x = 1 - my_x
        peer_y = 1 - my_y

        barrier = pltpu.get_barrier_semaphore()
        pl.semaphore_signal(
            barrier, inc=1, device_id=(peer_x, my_y),
            device_id_type=pl.DeviceIdType.MESH,
        )
        pl.semaphore_signal(
            barrier, inc=1, device_id=(my_x, peer_y),
            device_id_type=pl.DeviceIdType.MESH,
        )
        pl.semaphore_wait(barrier, 2)

        def chunk(c):
            return c // SPLIT, (c % SPLIT) * R

        def dot_t(a_t):
            return lax.dot_general(
                a_t.astype(jnp.bfloat16), wo_bf[...],
                dimension_numbers=(((0,), (0,)), ((), ())),
                preferred_element_type=jnp.float32,
            )

        half = N // 2
        wo_bf[:, :half] = wo_ref[:, :half].astype(jnp.bfloat16)
        b0, r0 = chunk(0)
        a0 = o_ref[
            b0, :, pl.ds(peer_x * S_out + my_y * Q + r0, R)
        ].astype(jnp.bfloat16)
        x_send[0, :, :half] = lax.dot_general(
            a0, wo_bf[:, :half],
            dimension_numbers=(((0,), (0,)), ((), ())),
            preferred_element_type=jnp.float32,
        ).astype(jnp.bfloat16)
        wo_bf[:, half:] = wo_ref[:, half:].astype(jnp.bfloat16)
        x_send[0, :, half:] = lax.dot_general(
            a0, wo_bf[:, half:],
            dimension_numbers=(((0,), (0,)), ((), ())),
            preferred_element_type=jnp.float32,
        ).astype(jnp.bfloat16)

        x_rdmas = []
        for c in range(NCHUNK):
            b, r = chunk(c)
            if c > 0:
                x_send[c, :, :] = dot_t(
                    o_ref[b, :, pl.ds(peer_x * S_out + my_y * Q + r, R)]
                ).astype(jnp.bfloat16)
            rdma = pltpu.make_async_remote_copy(
                src_ref=x_send.at[c],
                dst_ref=x_recv.at[c],
                send_sem=x_send_sems.at[c],
                recv_sem=x_recv_sems.at[c],
                device_id=(peer_x, my_y),
                device_id_type=pl.DeviceIdType.MESH,
            )
            rdma.start()
            x_rdmas.append(rdma)

        for c in range(NCHUNK):
            b, r = chunk(c)
            out_ref[b, pl.ds(my_y * Q + r, R), :] = dot_t(
                o_ref[b, :, pl.ds(my_x * S_out + my_y * Q + r, R)]
            )

        y_rdmas = []
        for c in range(NCHUNK):
            b, r = chunk(c)
            x_rdmas[c].wait_recv()
            sl = pl.ds(my_y * Q + r, R)
            s = out_ref[b, sl, :] + x_recv[c, :, :].astype(jnp.float32)
            y_send[c, :, :] = s.astype(jnp.bfloat16)
            rdma = pltpu.make_async_remote_copy(
                src_ref=y_send.at[c],
                dst_ref=y_recv.at[c],
                send_sem=y_send_sems.at[c],
                recv_sem=y_recv_sems.at[c],
                device_id=(my_x, peer_y),
                device_id_type=pl.DeviceIdType.MESH,
            )
            rdma.start()
            y_rdmas.append(rdma)
            out_ref[b, sl, :] = s

        for c in range(NCHUNK):
            b, r = chunk(c)
            y_rdmas[c].wait_recv()
            out_ref[b, pl.ds(peer_y * Q + r, R), :] = (
                y_recv[c, :, :].astype(jnp.float32)
            )

        for c in range(NCHUNK):
            x_rdmas[c].wait_send()
            y_rdmas[c].wait_send()

    return pl.pallas_call(
        body,
        out_shape=jax.ShapeDtypeStruct((B, S_out, N), jnp.float32),
        in_specs=[
            pl.BlockSpec(memory_space=pltpu.VMEM),
            pl.BlockSpec(memory_space=pltpu.VMEM),
        ],
        out_specs=pl.BlockSpec(memory_space=pltpu.VMEM),
        scratch_shapes=[
            pltpu.VMEM((K, N), jnp.bfloat16),
            pltpu.VMEM((NCHUNK, R, N), jnp.bfloat16),
            pltpu.VMEM((NCHUNK, R, N), jnp.bfloat16),
            pltpu.VMEM((NCHUNK, R, N), jnp.bfloat16),
            pltpu.VMEM((NCHUNK, R, N), jnp.bfloat16),
            pltpu.SemaphoreType.DMA((NCHUNK,)),
            pltpu.SemaphoreType.DMA((NCHUNK,)),
            pltpu.SemaphoreType.DMA((NCHUNK,)),
            pltpu.SemaphoreType.DMA((NCHUNK,)),
        ],
        compiler_params=pltpu.CompilerParams(collective_id=0),
    )(OT, Wo)
